# initial kernel scaffold (unmeasured)
import jax
import jax.numpy as jnp
from jax import lax
from jax.experimental import pallas as pl
from jax.experimental.pallas import tpu as pltpu

N_DEV = 16


def kernel(x, Wq, K_ext, V_ext, Wo):
    B, Sq, Din = x.shape
    _, HD = Wq.shape
    Bg, Skv, Hq, Dh = K_ext.shape
    Hloc = HD // Dh
    Dout = Wo.shape[1]
    BSq = B * Sq

    def body(x_ref, wq_ref, k_hbm, v_hbm, wo_ref, out_ref,
             wq_all, wo_all, k_loc, v_loc, ctx_s, acc,
             qsend, qrecv, osend, orecv, kv_sems):
        my = lax.axis_index("i")
        left = lax.rem(my + N_DEV - 1, N_DEV)
        right = lax.rem(my + 1, N_DEV)

        kcp = pltpu.make_async_copy(
            k_hbm.at[pl.ds(B * my, B)], k_loc, kv_sems.at[0])
        vcp = pltpu.make_async_copy(
            v_hbm.at[pl.ds(B * my, B)], v_loc, kv_sems.at[1])
        kcp.start()
        vcp.start()

        wq_all[0, :, :] = wq_ref[:, :]
        wo_all[0, :, :] = wo_ref[:, :]

        bar = pltpu.get_barrier_semaphore()
        for nbr in (left, right):
            pl.semaphore_signal(
                bar, inc=1, device_id=(nbr,),
                device_id_type=pl.DeviceIdType.MESH)
        pl.semaphore_wait(bar, 2)

        for h in range(N_DEV - 1):
            rq = pltpu.make_async_remote_copy(
                src_ref=wq_all.at[h], dst_ref=wq_all.at[h + 1],
                send_sem=qsend.at[h], recv_sem=qrecv.at[h],
                device_id=(right,), device_id_type=pl.DeviceIdType.MESH)
            ro = pltpu.make_async_remote_copy(
                src_ref=wo_all.at[h], dst_ref=wo_all.at[h + 1],
                send_sem=osend.at[h], recv_sem=orecv.at[h],
                device_id=(right,), device_id_type=pl.DeviceIdType.MESH)
            rq.start()
            ro.start()
            rq.wait()
            ro.wait()

        x2 = x_ref[...].reshape(BSq, Din)
        kcp.wait()
        vcp.wait()
        acc[...] = jnp.zeros((BSq, Dout), jnp.float32)

        def chunk_body(k, carry):
            j = lax.rem(my - k + N_DEV, N_DEV)
            ho = j * Hloc
            wq_k = wq_all[pl.ds(k, 1)].reshape(Din, HD)
            q = jnp.dot(x2, wq_k, preferred_element_type=jnp.float32)
            for b in range(B):
                for hh in range(Hloc):
                    qbh = q[b * Sq:(b + 1) * Sq, hh * Dh:(hh + 1) * Dh]
                    kbh = k_loc[b, :, pl.ds(ho + hh, 1), :].reshape(Skv, Dh)
                    vbh = v_loc[b, :, pl.ds(ho + hh, 1), :].reshape(Skv, Dh)
                    s = lax.dot_general(
                        qbh, kbh, (((1,), (1,)), ((), ())),
                        preferred_element_type=jnp.float32) * 0.125
                    m = jnp.max(s, axis=-1, keepdims=True)
                    w = jnp.exp(s - m)
                    w = w / jnp.sum(w, axis=-1, keepdims=True)
                    ctx_s[b * Sq:(b + 1) * Sq, hh * Dh:(hh + 1) * Dh] = (
                        jnp.dot(w, vbh, preferred_element_type=jnp.float32))
            wo_k = wo_all[pl.ds(k, 1)].reshape(HD, Dout)
            acc[...] = acc[...] + jnp.dot(
                ctx_s[...], wo_k, preferred_element_type=jnp.float32)
            return carry

        lax.fori_loop(0, N_DEV, chunk_body, None)
        out_ref[...] = acc[...].reshape(B, Sq, Dout)

    return pl.pallas_call(
        body,
        out_shape=jax.ShapeDtypeStruct((B, Sq, Dout), jnp.float32),
        in_specs=[
            pl.BlockSpec(memory_space=pltpu.VMEM),
            pl.BlockSpec(memory_space=pltpu.VMEM),
            pl.BlockSpec(memory_space=pltpu.ANY),
            pl.BlockSpec(memory_space=pltpu.ANY),
            pl.BlockSpec(memory_space=pltpu.VMEM),
        ],
        out_specs=pl.BlockSpec(memory_space=pltpu.VMEM),
        scratch_shapes=[
            pltpu.VMEM((N_DEV, Din, HD), jnp.float32),
            pltpu.VMEM((N_DEV, HD, Dout), jnp.float32),
            pltpu.VMEM((B, Skv, Hq, Dh), jnp.float32),
            pltpu.VMEM((B, Skv, Hq, Dh), jnp.float32),
            pltpu.VMEM((BSq, HD), jnp.float32),
            pltpu.VMEM((BSq, Dout), jnp.float32),
            pltpu.SemaphoreType.DMA((N_DEV - 1,)),
            pltpu.SemaphoreType.DMA((N_DEV - 1,)),
            pltpu.SemaphoreType.DMA((N_DEV - 1,)),
            pltpu.SemaphoreType.DMA((N_DEV - 1,)),
            pltpu.SemaphoreType.DMA((2,)),
        ],
        compiler_params=pltpu.CompilerParams(collective_id=0),
    )(x, Wq, K_ext, V_ext, Wo)


# baseline (device time: 482052 ns/iter reference)
import jax
import jax.numpy as jnp
from jax import lax
from jax.experimental import pallas as pl
from jax.experimental.pallas import tpu as pltpu

N_DEV = 16


def kernel(x, Wq, K_ext, V_ext, Wo):
    B, Sq, Din = x.shape
    _, HD = Wq.shape
    Bg, Skv, Hq, Dh = K_ext.shape
    Hloc = HD // Dh
    Dout = Wo.shape[1]
    BSq = B * Sq

    def body(x_ref, wq_ref, k_hbm, v_hbm, wo_ref, out_ref,
             wq_all, wo_all, k_loc, v_loc, ctx_s, acc,
             qsend, qrecv, osend, orecv, kv_sems):
        my = lax.axis_index("i")
        left = lax.rem(my + N_DEV - 1, N_DEV)
        right = lax.rem(my + 1, N_DEV)

        kcp = pltpu.make_async_copy(
            k_hbm.at[pl.ds(B * my, B)], k_loc, kv_sems.at[0])
        vcp = pltpu.make_async_copy(
            v_hbm.at[pl.ds(B * my, B)], v_loc, kv_sems.at[1])
        kcp.start()
        vcp.start()

        wq_all[0, :, :] = wq_ref[:, :]
        wo_all[0, :, :] = wo_ref[:, :]

        bar = pltpu.get_barrier_semaphore()
        for nbr in (left, right):
            pl.semaphore_signal(
                bar, inc=1, device_id=(nbr,),
                device_id_type=pl.DeviceIdType.MESH)
        pl.semaphore_wait(bar, 2)

        for h in range(N_DEV - 1):
            rq = pltpu.make_async_remote_copy(
                src_ref=wq_all.at[h], dst_ref=wq_all.at[h + 1],
                send_sem=qsend.at[h], recv_sem=qrecv.at[h],
                device_id=(right,), device_id_type=pl.DeviceIdType.MESH)
            ro = pltpu.make_async_remote_copy(
                src_ref=wo_all.at[h], dst_ref=wo_all.at[h + 1],
                send_sem=osend.at[h], recv_sem=orecv.at[h],
                device_id=(right,), device_id_type=pl.DeviceIdType.MESH)
            rq.start()
            ro.start()
            rq.wait()
            ro.wait()

        x2 = x_ref[...].reshape(BSq, Din)
        kcp.wait()
        vcp.wait()
        acc[...] = jnp.zeros((BSq, Dout), jnp.float32)

        def chunk_body(k, carry):
            j = lax.rem(my - k + N_DEV, N_DEV)
            ho = j * Hloc
            wq_k = wq_all[pl.ds(k, 1)].reshape(Din, HD)
            q = jnp.dot(x2, wq_k, preferred_element_type=jnp.float32)
            for b in range(B):
                for hh in range(Hloc):
                    qbh = q[b * Sq:(b + 1) * Sq, hh * Dh:(hh + 1) * Dh]
                    kbh = k_loc[b, :, pl.ds(ho + hh, 1), :].reshape(Skv, Dh)
                    vbh = v_loc[b, :, pl.ds(ho + hh, 1), :].reshape(Skv, Dh)
                    s = lax.dot_general(
                        qbh, kbh, (((1,), (1,)), ((), ())),
                        preferred_element_type=jnp.float32) * 0.125
                    m = jnp.max(s, axis=-1, keepdims=True)
                    w = jnp.exp(s - m)
                    w = w / jnp.sum(w, axis=-1, keepdims=True)
                    ctx_s[b * Sq:(b + 1) * Sq, hh * Dh:(hh + 1) * Dh] = (
                        jnp.dot(w, vbh, preferred_element_type=jnp.float32))
            wo_k = wo_all[pl.ds(k, 1)].reshape(HD, Dout)
            acc[...] = acc[...] + jnp.dot(
                ctx_s[...], wo_k, preferred_element_type=jnp.float32)
            return carry

        lax.fori_loop(0, N_DEV, chunk_body, None)
        out_ref[...] = acc[...].reshape(B, Sq, Dout)

    return pl.pallas_call(
        body,
        out_shape=jax.ShapeDtypeStruct((B, Sq, Dout), jnp.float32),
        in_specs=[
            pl.BlockSpec(memory_space=pltpu.VMEM),
            pl.BlockSpec(memory_space=pltpu.VMEM),
            pl.BlockSpec(memory_space=pl.ANY),
            pl.BlockSpec(memory_space=pl.ANY),
            pl.BlockSpec(memory_space=pltpu.VMEM),
        ],
        out_specs=pl.BlockSpec(memory_space=pltpu.VMEM),
        scratch_shapes=[
            pltpu.VMEM((N_DEV, Din, HD), jnp.float32),
            pltpu.VMEM((N_DEV, HD, Dout), jnp.float32),
            pltpu.VMEM((B, Skv, Hq, Dh), jnp.float32),
            pltpu.VMEM((B, Skv, Hq, Dh), jnp.float32),
            pltpu.VMEM((BSq, HD), jnp.float32),
            pltpu.VMEM((BSq, Dout), jnp.float32),
            pltpu.SemaphoreType.DMA((N_DEV - 1,)),
            pltpu.SemaphoreType.DMA((N_DEV - 1,)),
            pltpu.SemaphoreType.DMA((N_DEV - 1,)),
            pltpu.SemaphoreType.DMA((N_DEV - 1,)),
            pltpu.SemaphoreType.DMA((2,)),
        ],
        compiler_params=pltpu.CompilerParams(
            collective_id=0, vmem_limit_bytes=56 * 1024 * 1024),
    )(x, Wq, K_ext, V_ext, Wo)


# device time: 31958 ns/iter; 15.0839x vs baseline; 15.0839x over previous
import jax
import jax.numpy as jnp
from jax import lax
from jax.experimental import pallas as pl
from jax.experimental.pallas import tpu as pltpu

N_DEV = 16
NA = 9
NB = 8


def kernel(x, Wq, K_ext, V_ext, Wo):
    B, Sq, Din = x.shape
    _, HD = Wq.shape
    Bg, Skv, Hq, Dh = K_ext.shape
    Hloc = HD // Dh
    Dout = Wo.shape[1]
    BSq = B * Sq
    bf16 = jnp.bfloat16

    def body(x_ref, wq_ref, k_hbm, v_hbm, wo_ref, out_ref,
             aq, ao, bq, bo, k_loc, v_loc, ctx_s, acc,
             aq_s, aq_r, ao_s, ao_r, bq_s, bq_r, bo_s, bo_r, kv_sems):
        my = lax.axis_index("i")
        left = lax.rem(my + N_DEV - 1, N_DEV)
        right = lax.rem(my + 1, N_DEV)

        kcp = pltpu.make_async_copy(
            k_hbm.at[pl.ds(B * my, B)], k_loc, kv_sems.at[0])
        vcp = pltpu.make_async_copy(
            v_hbm.at[pl.ds(B * my, B)], v_loc, kv_sems.at[1])
        kcp.start()
        vcp.start()

        wqb = wq_ref[...].astype(bf16)
        wob = wo_ref[...].astype(bf16)
        aq[0, :, :] = wqb
        ao[0, :, :] = wob
        bq[0, :, :] = wqb
        bo[0, :, :] = wob

        bar = pltpu.get_barrier_semaphore()
        for nbr in (left, right):
            pl.semaphore_signal(
                bar, inc=1, device_id=(nbr,),
                device_id_type=pl.DeviceIdType.MESH)
        pl.semaphore_wait(bar, 2)

        x2 = x_ref[...].reshape(BSq, Din).astype(bf16)
        kcp.wait()
        vcp.wait()
        acc[...] = jnp.zeros((BSq, Dout), jnp.float32)

        def mk_a(h):
            rq = pltpu.make_async_remote_copy(
                src_ref=aq.at[h], dst_ref=aq.at[h + 1],
                send_sem=aq_s.at[h], recv_sem=aq_r.at[h],
                device_id=(right,), device_id_type=pl.DeviceIdType.MESH)
            ro = pltpu.make_async_remote_copy(
                src_ref=ao.at[h], dst_ref=ao.at[h + 1],
                send_sem=ao_s.at[h], recv_sem=ao_r.at[h],
                device_id=(right,), device_id_type=pl.DeviceIdType.MESH)
            return rq, ro

        def mk_b(h):
            rq = pltpu.make_async_remote_copy(
                src_ref=bq.at[h], dst_ref=bq.at[h + 1],
                send_sem=bq_s.at[h], recv_sem=bq_r.at[h],
                device_id=(left,), device_id_type=pl.DeviceIdType.MESH)
            ro = pltpu.make_async_remote_copy(
                src_ref=bo.at[h], dst_ref=bo.at[h + 1],
                send_sem=bo_s.at[h], recv_sem=bo_r.at[h],
                device_id=(left,), device_id_type=pl.DeviceIdType.MESH)
            return rq, ro

        def compute(wq_k, wo_k, j):
            ho = j * Hloc
            q = jnp.dot(x2, wq_k, preferred_element_type=jnp.float32)
            for b in range(B):
                for hh in range(Hloc):
                    qbh = q[b * Sq:(b + 1) * Sq,
                            hh * Dh:(hh + 1) * Dh].astype(bf16)
                    kbh = k_loc[b, :, pl.ds(ho + hh, 1), :].reshape(
                        Skv, Dh).astype(bf16)
                    vbh = v_loc[b, :, pl.ds(ho + hh, 1), :].reshape(
                        Skv, Dh).astype(bf16)
                    s = lax.dot_general(
                        qbh, kbh, (((1,), (1,)), ((), ())),
                        preferred_element_type=jnp.float32) * 0.125
                    m = jnp.max(s, axis=-1, keepdims=True)
                    w = jnp.exp(s - m)
                    w = (w / jnp.sum(w, axis=-1, keepdims=True)).astype(bf16)
                    ctx_s[b * Sq:(b + 1) * Sq, hh * Dh:(hh + 1) * Dh] = (
                        jnp.dot(w, vbh,
                                preferred_element_type=jnp.float32)
                        .astype(bf16))
            acc[...] = acc[...] + jnp.dot(
                ctx_s[...], wo_k, preferred_element_type=jnp.float32)

        def compute_a(h):
            compute(aq[pl.ds(h, 1)].reshape(Din, HD),
                    ao[pl.ds(h, 1)].reshape(HD, Dout),
                    lax.rem(my - h + N_DEV, N_DEV))

        def compute_b(h):
            compute(bq[pl.ds(h, 1)].reshape(Din, HD),
                    bo[pl.ds(h, 1)].reshape(HD, Dout),
                    lax.rem(my + h, N_DEV))

        a0q, a0o = mk_a(0)
        b0q, b0o = mk_b(0)
        a0q.start(); a0o.start(); b0q.start(); b0o.start()
        compute_a(0)
        a0q.wait(); a0o.wait(); b0q.wait(); b0o.wait()

        def hop(h, carry):
            raq, rao = mk_a(h)
            rbq, rbo = mk_b(h)
            raq.start(); rao.start(); rbq.start(); rbo.start()
            compute_a(h)
            compute_b(h)
            raq.wait(); rao.wait(); rbq.wait(); rbo.wait()
            return carry

        lax.fori_loop(1, NB - 1, hop, None)

        a7q, a7o = mk_a(NA - 2)
        a7q.start(); a7o.start()
        compute_a(NA - 2)
        compute_b(NB - 1)
        a7q.wait(); a7o.wait()

        compute_a(NA - 1)

        out_ref[...] = acc[...].reshape(B, Sq, Dout)

    return pl.pallas_call(
        body,
        out_shape=jax.ShapeDtypeStruct((B, Sq, Dout), jnp.float32),
        in_specs=[
            pl.BlockSpec(memory_space=pltpu.VMEM),
            pl.BlockSpec(memory_space=pltpu.VMEM),
            pl.BlockSpec(memory_space=pl.ANY),
            pl.BlockSpec(memory_space=pl.ANY),
            pl.BlockSpec(memory_space=pltpu.VMEM),
        ],
        out_specs=pl.BlockSpec(memory_space=pltpu.VMEM),
        scratch_shapes=[
            pltpu.VMEM((NA, Din, HD), bf16),
            pltpu.VMEM((NA, HD, Dout), bf16),
            pltpu.VMEM((NB, Din, HD), bf16),
            pltpu.VMEM((NB, HD, Dout), bf16),
            pltpu.VMEM((B, Skv, Hq, Dh), jnp.float32),
            pltpu.VMEM((B, Skv, Hq, Dh), jnp.float32),
            pltpu.VMEM((BSq, HD), bf16),
            pltpu.VMEM((BSq, Dout), jnp.float32),
            pltpu.SemaphoreType.DMA((NA - 1,)),
            pltpu.SemaphoreType.DMA((NA - 1,)),
            pltpu.SemaphoreType.DMA((NA - 1,)),
            pltpu.SemaphoreType.DMA((NA - 1,)),
            pltpu.SemaphoreType.DMA((NB - 1,)),
            pltpu.SemaphoreType.DMA((NB - 1,)),
            pltpu.SemaphoreType.DMA((NB - 1,)),
            pltpu.SemaphoreType.DMA((NB - 1,)),
            pltpu.SemaphoreType.DMA((2,)),
        ],
        compiler_params=pltpu.CompilerParams(
            collective_id=0, vmem_limit_bytes=56 * 1024 * 1024),
    )(x, Wq, K_ext, V_ext, Wo)
